# baseline (speedup 1.0000x reference)
import functools

import jax
import jax.numpy as jnp
from jax import lax
from jax.experimental import pallas as pl
from jax.experimental.pallas import tpu as pltpu
from jax.experimental.pallas import tpu_sc as plsc

CH = 128
K = 8
SUP = CH * K


def _sc_gather(table, idx2d, n_rows):
    info = plsc.get_sparse_core_info()
    nc, ns = info.num_cores, info.num_subcores
    nw = nc * ns
    d = table.shape[1]
    rows_per_w = n_rows // nw
    n_sup = rows_per_w // SUP

    mesh = plsc.VectorSubcoreMesh(core_axis_name="c", subcore_axis_name="s")

    @functools.partial(
        pl.kernel,
        out_type=jax.ShapeDtypeStruct((n_rows, d), jnp.float32),
        mesh=mesh,
        scratch_types=[
            pltpu.VMEM((K, CH), jnp.int32),
            pltpu.VMEM((SUP, d), jnp.float32),
            pltpu.SemaphoreType.DMA,
        ],
        compiler_params=pltpu.CompilerParams(use_tc_tiling_on_sc=False),
    )
    def gather_kernel(table_hbm, idx_hbm, out_hbm, idx_v, rows_v, gsem):
        wid = lax.axis_index("s") * nc + lax.axis_index("c")
        row0 = wid * rows_per_w
        irow0 = wid * (rows_per_w // CH)

        def super_iter(ob, _):
            pltpu.sync_copy(idx_hbm.at[pl.ds(irow0 + ob * K, K)], idx_v)
            copies = []
            for j in range(K):
                copies.append(pltpu.async_copy(
                    table_hbm.at[idx_v.at[j]],
                    rows_v.at[pl.ds(j * CH, CH)],
                    gsem,
                ))
            for c in copies:
                c.wait()
            pltpu.sync_copy(rows_v, out_hbm.at[pl.ds(row0 + ob * SUP, SUP)])
            return _

        lax.fori_loop(0, n_sup, super_iter, 0)

    return gather_kernel(table, idx2d)


def _tc_project(emb, Wt, b2d):
    n, d = emb.shape
    d_out = Wt.shape[1]
    blk = 2048

    def body(e_ref, w_ref, b_ref, o_ref):
        o_ref[...] = (
            jnp.dot(e_ref[...], w_ref[...], preferred_element_type=jnp.float32)
            + b_ref[...]
        )

    return pl.pallas_call(
        body,
        grid=(n // blk,),
        in_specs=[
            pl.BlockSpec((blk, d), lambda i: (i, 0)),
            pl.BlockSpec((d, d_out), lambda i: (0, 0)),
            pl.BlockSpec((1, d_out), lambda i: (0, 0)),
        ],
        out_specs=pl.BlockSpec((blk, d_out), lambda i: (i, 0)),
        out_shape=jax.ShapeDtypeStruct((n, d_out), jnp.float32),
    )(emb, Wt, b2d)


def kernel(x, table, W, b):
    batch, seq = x.shape
    n = batch * seq
    idx2d = x.reshape(n // CH, CH)
    emb = _sc_gather(table, idx2d, n)
    out = _tc_project(emb, W.T, b.reshape(1, -1))
    return out.reshape(batch, seq, -1)


# speedup vs baseline: 1.7989x; 1.7989x over previous
import functools

import jax
import jax.numpy as jnp
from jax import lax
from jax.experimental import pallas as pl
from jax.experimental.pallas import tpu as pltpu
from jax.experimental.pallas import tpu_sc as plsc

CH = 128
K = 8
SUP = CH * K
BLK = 4096
HALF = 512000


def _tc_project_table(tableT, V, b2d):
    d, vocab = tableT.shape
    d_out = V.shape[1]

    def body(tlo_ref, thi_ref, v_ref, b_ref, o_ref):
        def proj(t):
            return lax.dot_general(
                t, v_ref[...],
                dimension_numbers=(((0,), (0,)), ((), ())),
                preferred_element_type=jnp.float32,
            ) + b_ref[...]
        o_ref[...] = jnp.concatenate(
            [proj(tlo_ref[...]), proj(thi_ref[...])], axis=1)

    nblk = HALF // BLK
    last = (vocab - 1) // BLK
    return pl.pallas_call(
        body,
        grid=(nblk,),
        in_specs=[
            pl.BlockSpec((d, BLK), lambda i: (0, i)),
            pl.BlockSpec((d, BLK), lambda i: (0, jnp.minimum(i + nblk, last))),
            pl.BlockSpec((d, d_out), lambda i: (0, 0)),
            pl.BlockSpec((1, d_out), lambda i: (0, 0)),
        ],
        out_specs=pl.BlockSpec((BLK, 2 * d_out), lambda i: (i, 0)),
        out_shape=jax.ShapeDtypeStruct((HALF, 2 * d_out), jnp.float32),
    )(tableT, tableT, V, b2d)


def _sc_gather(p_rows, idx2d, n_rows):
    info = plsc.get_sparse_core_info()
    nc, ns = info.num_cores, info.num_subcores
    nw = nc * ns
    d = p_rows.shape[1]
    rows_per_w = n_rows // nw
    n_sup = rows_per_w // SUP

    mesh = plsc.VectorSubcoreMesh(core_axis_name="c", subcore_axis_name="s")

    @functools.partial(
        pl.kernel,
        out_type=jax.ShapeDtypeStruct((n_rows, d), jnp.float32),
        mesh=mesh,
        scratch_types=[
            pltpu.VMEM((K, CH), jnp.int32),
            pltpu.VMEM((SUP, d), jnp.float32),
            pltpu.SemaphoreType.DMA,
        ],
        compiler_params=pltpu.CompilerParams(use_tc_tiling_on_sc=False),
    )
    def gather_kernel(table_hbm, idx_hbm, out_hbm, idx_v, rows_v, gsem):
        wid = lax.axis_index("s") * nc + lax.axis_index("c")
        row0 = wid * rows_per_w
        irow0 = wid * (rows_per_w // CH)

        def super_iter(ob, _):
            pltpu.sync_copy(idx_hbm.at[pl.ds(irow0 + ob * K, K)], idx_v)
            copies = []
            for j in range(K):
                copies.append(pltpu.async_copy(
                    table_hbm.at[idx_v.at[j]],
                    rows_v.at[pl.ds(j * CH, CH)],
                    gsem,
                ))
            for c in copies:
                c.wait()
            pltpu.sync_copy(rows_v, out_hbm.at[pl.ds(row0 + ob * SUP, SUP)])
            return _

        lax.fori_loop(0, n_sup, super_iter, 0)

    return gather_kernel(p_rows, idx2d)


def kernel(x, table, W, b):
    batch, seq = x.shape
    n = batch * seq
    vocab, d = table.shape
    xf = x.reshape(n)
    idx2d = jnp.where(xf < HALF, 2 * xf, 2 * (xf - HALF) + 1).reshape(
        n // CH, CH)
    p2 = _tc_project_table(table.T, W.T, b.reshape(1, -1))
    p_rows = p2.reshape(2 * HALF, d)
    emb = _sc_gather(p_rows, idx2d, n)
    return emb.reshape(batch, seq, d)
